# baseline (speedup 1.0000x reference)
import functools

import jax
import jax.numpy as jnp
from jax import lax
from jax.experimental import pallas as pl
from jax.experimental.pallas import tpu as pltpu
from jax.experimental.pallas import tpu_sc as plsc

N_USERS = 1000000
N_ITEMS = 1000000
D = 64
B = 16384

NC = 2
NS = 16
NW = NC * NS
BPW = B // NW
CHUNK = 128
NCHUNK = BPW // CHUNK


@functools.partial(
    pl.kernel,
    out_type=jax.ShapeDtypeStruct((B,), jnp.float32),
    mesh=plsc.VectorSubcoreMesh(core_axis_name="c", subcore_axis_name="s"),
    compiler_params=pltpu.CompilerParams(
        needs_layout_passes=False, use_tc_tiling_on_sc=False),
    scratch_types=[
        pltpu.VMEM((NCHUNK, CHUNK), jnp.int32),
        pltpu.VMEM((NCHUNK, CHUNK), jnp.int32),
        pltpu.VMEM((BPW, D), jnp.float32),
        pltpu.VMEM((BPW, D), jnp.float32),
        pltpu.VMEM((BPW,), jnp.float32),
        pltpu.VMEM((BPW,), jnp.float32),
        pltpu.VMEM((BPW,), jnp.float32),
        pltpu.SemaphoreType.DMA,
    ],
)
def _lfm_sc(users2d, items2d, ue, ie, ub, ib, out,
            uidx_v, iidx_v, urows_v, irows_v, ub_v, ib_v, out_v, sem):
    wid = lax.axis_index("s") * NC + lax.axis_index("c")
    base = wid * BPW

    pltpu.sync_copy(users2d.at[pl.ds(wid * NCHUNK, NCHUNK)], uidx_v)
    pltpu.sync_copy(items2d.at[pl.ds(wid * NCHUNK, NCHUNK)], iidx_v)

    cps = []
    for c in range(NCHUNK):
        rows = pl.ds(c * CHUNK, CHUNK)
        cps.append(pltpu.async_copy(ue.at[uidx_v.at[c]], urows_v.at[rows], sem))
        cps.append(pltpu.async_copy(ie.at[iidx_v.at[c]], irows_v.at[rows], sem))
        cps.append(pltpu.async_copy(ub.at[uidx_v.at[c]], ub_v.at[rows], sem))
        cps.append(pltpu.async_copy(ib.at[iidx_v.at[c]], ib_v.at[rows], sem))
    for cp in cps:
        cp.wait()

    lane = lax.iota(jnp.int32, 16)

    def group(g, carry):
        tot = jnp.zeros((16,), jnp.float32)
        for l in range(16):
            r = g * 16 + l
            acc = (urows_v[r, pl.ds(0, 16)] * irows_v[r, pl.ds(0, 16)]
                   + urows_v[r, pl.ds(16, 16)] * irows_v[r, pl.ds(16, 16)])
            acc = acc + (urows_v[r, pl.ds(32, 16)] * irows_v[r, pl.ds(32, 16)]
                         + urows_v[r, pl.ds(48, 16)] * irows_v[r, pl.ds(48, 16)])
            tot = jnp.where(lane == l, jnp.sum(acc), tot)
        rows16 = pl.ds(g * 16, 16)
        out_v[rows16] = tot + ub_v[rows16] + ib_v[rows16]
        return carry

    lax.fori_loop(0, BPW // 16, group, 0)

    pltpu.sync_copy(out_v, out.at[pl.ds(base, BPW)])


def kernel(users, items, user_embeddings, item_embeddings, user_biases, item_biases):
    users2d = users.astype(jnp.int32).reshape(B // CHUNK, CHUNK)
    items2d = items.astype(jnp.int32).reshape(B // CHUNK, CHUNK)
    return _lfm_sc(users2d, items2d, user_embeddings, item_embeddings,
                   user_biases.reshape(N_USERS), item_biases.reshape(N_ITEMS))
